# baseline (speedup 1.0000x reference)
import functools

import jax
import jax.numpy as jnp
from jax import lax
from jax.experimental import pallas as pl
from jax.experimental.pallas import tpu as pltpu
from jax.experimental.pallas import tpu_sc as plsc

B = 20000
N_FEAT_ROWS = 100000
S1 = 26
D = 128
WPAD = 32
L = 16

NC = 2
NS = 16
NW = NC * NS

R = 32
NCHUNK = B // R

BSZ = 512
BP = 20480


def _weights_body(aT_ref, w_ref):
    a = aT_ref[...]
    eq = a[None, :, :] == a[:, None, :]
    jj = lax.broadcasted_iota(jnp.int32, (S1, S1, 1), 0)
    kk = lax.broadcasted_iota(jnp.int32, (S1, S1, 1), 1)
    dup = jnp.any(eq & (kk < jj), axis=1)
    valid = (~dup).astype(jnp.float32)
    num = jnp.sum(valid, axis=0, keepdims=True)
    w = valid / num
    wp = jnp.concatenate(
        [w, jnp.zeros((WPAD - S1, BSZ), jnp.float32)], axis=0)
    w_ref[...] = jnp.swapaxes(wp, 0, 1)


_tc_weights = pl.pallas_call(
    _weights_body,
    grid=(BP // BSZ,),
    in_specs=[pl.BlockSpec((S1, BSZ), lambda i: (0, i))],
    out_specs=pl.BlockSpec((BSZ, WPAD), lambda i: (i, 0)),
    out_shape=jax.ShapeDtypeStruct((BP, WPAD), jnp.float32),
)


T = (NCHUNK + NW - 1) // NW
HALF = 104
assert (R * S1) % HALF == 0


def _sc_body(feat_hbm, idxf_hbm, w_hbm, out_hbm,
             idx0, idx1, w0, w1, rows0, rows1, out0, out1,
             semg0, semg1, semi0, semi1, semw0, semw1, semo0, semo1):
    wid = lax.axis_index("s") * NC + lax.axis_index("c")
    idx = (idx0, idx1)
    wv_ = (w0, w1)
    rows = (rows0, rows1)
    outv = (out0, out1)
    semg = (semg0, semg1)
    semi = (semi0, semi1)
    semw = (semw0, semw1)
    semo = (semo0, semo1)

    def c_of(t):
        return jnp.minimum(wid + t * NW, NCHUNK - 1)

    def issue_idx(t, p):
        pltpu.async_copy(
            idxf_hbm.at[pl.ds(c_of(t) * R * S1, R * S1)], idx[p], semi[p])

    def wait_idx(p):
        pltpu.make_async_copy(
            idxf_hbm.at[pl.ds(0, R * S1)], idx[p], semi[p]).wait()

    def issue_w(t, p):
        pltpu.async_copy(w_hbm.at[pl.ds(c_of(t) * R, R)], wv_[p], semw[p])

    def wait_w(p):
        pltpu.make_async_copy(w_hbm.at[pl.ds(0, R)], wv_[p], semw[p]).wait()

    def issue_gather(p):
        for s in range(0, R * S1, HALF):
            pltpu.async_copy(feat_hbm.at[idx[p].at[pl.ds(s, HALF)]],
                             rows[p].at[pl.ds(s, HALF)], semg[p])

    def wait_gather(p):
        pltpu.make_async_copy(
            feat_hbm.at[pl.ds(0, R * S1)], rows[p], semg[p]).wait()

    def issue_out(t, p):
        pltpu.async_copy(outv[p], out_hbm.at[pl.ds(c_of(t) * R, R)], semo[p])

    def wait_out(p):
        pltpu.make_async_copy(
            outv[p], out_hbm.at[pl.ds(0, R)], semo[p]).wait()

    def compute(p):
        rows_v = rows[p]
        w_v = wv_[p]
        out_v = outv[p]
        for r in range(R):
            wlo = w_v[r, pl.ds(0, L)]
            whi = w_v[r, pl.ds(L, L)]
            def j_body(j, accs, wlo=wlo, whi=whi, r=r):
                jm = jnp.full((L,), j & (L - 1), jnp.int32)
                wvec = jnp.where(j < L,
                                 wlo.at[jm].get(mode="promise_in_bounds"),
                                 whi.at[jm].get(mode="promise_in_bounds"))
                row = r * S1 + j
                new = []
                for m in range(D // (2 * L)):
                    ab32 = rows_v[row, pl.ds(L * m, L)]
                    a = plsc.bitcast(ab32 << 16, jnp.float32)
                    b = plsc.bitcast(
                        ab32 & jnp.int32(-65536), jnp.float32)
                    new.append(accs[2 * m] + wvec * a)
                    new.append(accs[2 * m + 1] + wvec * b)
                return tuple(new)
            accs = lax.fori_loop(
                0, S1, j_body,
                tuple(jnp.zeros((L,), jnp.float32) for _ in range(D // L)))
            for k in range(D // L):
                out_v[r, pl.ds(k * L, L)] = accs[k]

    def step(t, p, first, second):
        q = 1 - p
        wait_gather(p)
        wait_idx(q)
        issue_gather(q)
        issue_idx(t + 2, p)
        if not (first or second):
            wait_out(p)
        wait_w(p)
        compute(p)
        issue_out(t, p)
        issue_w(t + 2, p)

    issue_idx(0, 0)
    issue_idx(1, 1)
    issue_w(0, 0)
    issue_w(1, 1)
    wait_idx(0)
    issue_gather(0)

    step(0, 0, True, False)
    step(1, 1, False, True)

    def pair_body(u, carry):
        t = 2 + 2 * u
        step(t, 0, False, False)
        step(t + 1, 1, False, False)
        return carry

    lax.fori_loop(0, (T - 2) // 2, pair_body, 0)
    if T % 2 == 1:
        step(T - 1, 0, False, False)

    pl_ = (T - 1) % 2
    ql_ = 1 - pl_
    wait_gather(ql_)
    wait_idx(pl_)
    wait_w(pl_)
    wait_out(pl_)
    wait_out(ql_)


@functools.cache
def _sc_aggregate():
    return functools.partial(
        pl.kernel,
        out_type=jax.ShapeDtypeStruct((B, D), jnp.float32),
        mesh=plsc.VectorSubcoreMesh(
            core_axis_name="c", subcore_axis_name="s",
            num_cores=NC, num_subcores=NS),
        compiler_params=pltpu.CompilerParams(
            needs_layout_passes=False, use_tc_tiling_on_sc=False),
        scratch_types=(
            [pltpu.VMEM((R * S1,), jnp.int32)] * 2
            + [pltpu.VMEM((R, WPAD), jnp.float32)] * 2
            + [pltpu.VMEM((R * S1, D // 2), jnp.int32)] * 2
            + [pltpu.VMEM((R, D), jnp.float32)] * 2
            + [pltpu.SemaphoreType.DMA] * 8
        ),
    )(_sc_body)


_PERM = [0] * D
for _m in range(D // 32):
    for _i in range(L):
        _PERM[32 * _m + 2 * _i] = 32 * _m + _i
        _PERM[32 * _m + 2 * _i + 1] = 32 * _m + L + _i
_PERM = tuple(_PERM)


def kernel(features, nodes, neigh_idx):
    nodes = nodes.astype(jnp.int32)
    neigh_idx = neigh_idx.astype(jnp.int32)
    features = lax.bitcast_convert_type(
        features[:, jnp.array(_PERM, jnp.int32)]
        .astype(jnp.bfloat16).reshape(N_FEAT_ROWS, D // 2, 2),
        jnp.int32)
    all_idx = jnp.concatenate([neigh_idx, nodes[:, None]], axis=1)
    aT = jnp.concatenate(
        [all_idx.T, jnp.zeros((S1, BP - B), jnp.int32)], axis=1)
    w = _tc_weights(aT)
    idx_flat = all_idx.reshape(B * S1)
    return _sc_aggregate()(features, idx_flat, w)


# speedup vs baseline: 2.1990x; 2.1990x over previous
import functools

import jax
import jax.numpy as jnp
from jax import lax
from jax.experimental import pallas as pl
from jax.experimental.pallas import tpu as pltpu
from jax.experimental.pallas import tpu_sc as plsc

B = 20000
N_FEAT_ROWS = 100000
S1 = 26
D = 128
WPAD = 32
L = 16

NC = 2
NS = 16
NW = NC * NS

R = 32
NCHUNK = B // R

BSZ = 512
BP = 20480


def _weights_body(aT_ref, w_ref):
    a = aT_ref[...]
    eq = a[None, :, :] == a[:, None, :]
    jj = lax.broadcasted_iota(jnp.int32, (S1, S1, 1), 0)
    kk = lax.broadcasted_iota(jnp.int32, (S1, S1, 1), 1)
    dup = jnp.any(eq & (kk < jj), axis=1)
    valid = (~dup).astype(jnp.float32)
    num = jnp.sum(valid, axis=0, keepdims=True)
    w = valid / num
    wp = jnp.concatenate(
        [w, jnp.zeros((WPAD - S1, BSZ), jnp.float32)], axis=0)
    w_ref[...] = jnp.swapaxes(wp, 0, 1)


_tc_weights = pl.pallas_call(
    _weights_body,
    grid=(BP // BSZ,),
    in_specs=[pl.BlockSpec((S1, BSZ), lambda i: (0, i))],
    out_specs=pl.BlockSpec((BSZ, WPAD), lambda i: (i, 0)),
    out_shape=jax.ShapeDtypeStruct((BP, WPAD), jnp.float32),
)


T = (NCHUNK + NW - 1) // NW
HALF = 104
assert (R * S1) % HALF == 0


def _sc_body(feat_hbm, idxf_hbm, w_hbm, out_hbm,
             idx0, idx1, w0, w1, rows0, rows1, out0, out1,
             semg0, semg1, semi0, semi1, semw0, semw1, semo0, semo1):
    wid = lax.axis_index("s") * NC + lax.axis_index("c")
    idx = (idx0, idx1)
    wv_ = (w0, w1)
    rows = (rows0, rows1)
    outv = (out0, out1)
    semg = (semg0, semg1)
    semi = (semi0, semi1)
    semw = (semw0, semw1)
    semo = (semo0, semo1)

    def c_of(t):
        return jnp.minimum(wid + t * NW, NCHUNK - 1)

    def issue_idx(t, p):
        pltpu.async_copy(
            idxf_hbm.at[pl.ds(c_of(t) * R * S1, R * S1)], idx[p], semi[p])

    def wait_idx(p):
        pltpu.make_async_copy(
            idxf_hbm.at[pl.ds(0, R * S1)], idx[p], semi[p]).wait()

    def issue_w(t, p):
        pltpu.async_copy(w_hbm.at[pl.ds(c_of(t) * R, R)], wv_[p], semw[p])

    def wait_w(p):
        pltpu.make_async_copy(w_hbm.at[pl.ds(0, R)], wv_[p], semw[p]).wait()

    def issue_gather(p):
        for s in range(0, R * S1, HALF):
            pltpu.async_copy(feat_hbm.at[idx[p].at[pl.ds(s, HALF)]],
                             rows[p].at[pl.ds(s, HALF)], semg[p])

    def wait_gather(p):
        pltpu.make_async_copy(
            feat_hbm.at[pl.ds(0, R * S1)], rows[p], semg[p]).wait()

    def issue_out(t, p):
        pltpu.async_copy(outv[p], out_hbm.at[pl.ds(c_of(t) * R, R)], semo[p])

    def wait_out(p):
        pltpu.make_async_copy(
            outv[p], out_hbm.at[pl.ds(0, R)], semo[p]).wait()

    def compute(p):
        rows_v = rows[p]
        w_v = wv_[p]
        out_v = outv[p]
        for r in range(R):
            wlo = w_v[r, pl.ds(0, L)]
            whi = w_v[r, pl.ds(L, L)]
            def j_body(j, accs, wlo=wlo, whi=whi, r=r):
                jm = jnp.full((L,), j & (L - 1), jnp.int32)
                wvec = jnp.where(j < L,
                                 wlo.at[jm].get(mode="promise_in_bounds"),
                                 whi.at[jm].get(mode="promise_in_bounds"))
                row = r * S1 + j
                new = []
                for m in range(D // (2 * L)):
                    ab32 = rows_v[row, pl.ds(L * m, L)]
                    a = plsc.bitcast(ab32 << 16, jnp.float32)
                    b = plsc.bitcast(
                        ab32 & jnp.int32(-65536), jnp.float32)
                    new.append(accs[2 * m] + wvec * a)
                    new.append(accs[2 * m + 1] + wvec * b)
                return tuple(new)
            accs = lax.fori_loop(
                0, S1, j_body,
                tuple(jnp.zeros((L,), jnp.float32) for _ in range(D // L)))
            for k in range(D // L):
                out_v[r, pl.ds(k * L, L)] = accs[k]

    def step(t, p, first, second):
        q = 1 - p
        wait_gather(p)
        wait_idx(q)
        issue_gather(q)
        issue_idx(t + 2, p)
        if not (first or second):
            wait_out(p)
        wait_w(p)
        compute(p)
        issue_out(t, p)
        issue_w(t + 2, p)

    issue_idx(0, 0)
    issue_idx(1, 1)
    issue_w(0, 0)
    issue_w(1, 1)
    wait_idx(0)
    issue_gather(0)

    step(0, 0, True, False)
    step(1, 1, False, True)

    def pair_body(u, carry):
        t = 2 + 2 * u
        step(t, 0, False, False)
        step(t + 1, 1, False, False)
        return carry

    lax.fori_loop(0, (T - 2) // 2, pair_body, 0)
    if T % 2 == 1:
        step(T - 1, 0, False, False)

    pl_ = (T - 1) % 2
    ql_ = 1 - pl_
    wait_gather(ql_)
    wait_idx(pl_)
    wait_w(pl_)
    wait_out(pl_)
    wait_out(ql_)


@functools.cache
def _sc_aggregate():
    return functools.partial(
        pl.kernel,
        out_type=jax.ShapeDtypeStruct((B, D), jnp.float32),
        mesh=plsc.VectorSubcoreMesh(
            core_axis_name="c", subcore_axis_name="s",
            num_cores=NC, num_subcores=NS),
        compiler_params=pltpu.CompilerParams(
            needs_layout_passes=False, use_tc_tiling_on_sc=False),
        scratch_types=(
            [pltpu.VMEM((R * S1,), jnp.int32)] * 2
            + [pltpu.VMEM((R, WPAD), jnp.float32)] * 2
            + [pltpu.VMEM((R * S1, D // 2), jnp.int32)] * 2
            + [pltpu.VMEM((R, D), jnp.float32)] * 2
            + [pltpu.SemaphoreType.DMA] * 8
        ),
    )(_sc_body)


_PERM = [0] * D
for _m in range(D // 32):
    for _i in range(L):
        _PERM[32 * _m + 2 * _i] = 32 * _m + _i
        _PERM[32 * _m + 2 * _i + 1] = 32 * _m + L + _i
_PERM = tuple(_PERM)


def kernel(features, nodes, neigh_idx):
    nodes = nodes.astype(jnp.int32)
    neigh_idx = neigh_idx.astype(jnp.int32)
    u = lax.bitcast_convert_type(features, jnp.uint32)
    r16 = (u + jnp.uint32(0x7FFF) + ((u >> 16) & jnp.uint32(1))) >> 16
    r3 = r16.reshape(N_FEAT_ROWS, D // 32, 32)
    packed = r3[:, :, :L] | (r3[:, :, L:] << 16)
    features = lax.bitcast_convert_type(
        packed.reshape(N_FEAT_ROWS, D // 2), jnp.int32)
    all_idx = jnp.concatenate([neigh_idx, nodes[:, None]], axis=1)
    aT = jnp.concatenate(
        [all_idx.T, jnp.zeros((S1, BP - B), jnp.int32)], axis=1)
    w = _tc_weights(aT)
    idx_flat = all_idx.reshape(B * S1)
    return _sc_aggregate()(features, idx_flat, w)


# speedup vs baseline: 3.3651x; 1.5303x over previous
import functools

import jax
import jax.numpy as jnp
from jax import lax
from jax.experimental import pallas as pl
from jax.experimental.pallas import tpu as pltpu
from jax.experimental.pallas import tpu_sc as plsc

B = 20000
S = 25
S1 = S + 1
D = 128
WPAD = 32
L = 16

NC = 2
NS = 16
NW = NC * NS

R = 16
NCHUNK = B // R

BSZ = 512
NBLK = (B + BSZ - 1) // BSZ
BP = NBLK * BSZ


def _weights_body(neigh_ref, nodes_ref, w_ref):
    nT = jnp.swapaxes(neigh_ref[...], 0, 1)
    self_row = jnp.swapaxes(nodes_ref[...], 0, 1)
    a = jnp.concatenate([nT, self_row], axis=0)
    eq = a[None, :, :] == a[:, None, :]
    jj = lax.broadcasted_iota(jnp.int32, (S1, S1, 1), 0)
    kk = lax.broadcasted_iota(jnp.int32, (S1, S1, 1), 1)
    dup = jnp.any(eq & (kk < jj), axis=1)
    valid = (~dup).astype(jnp.float32)
    num = jnp.sum(valid, axis=0, keepdims=True)
    w = valid / num
    wp = jnp.concatenate(
        [w, jnp.zeros((WPAD - S1, BSZ), jnp.float32)], axis=0)
    w_ref[...] = jnp.swapaxes(wp, 0, 1)


_tc_weights = pl.pallas_call(
    _weights_body,
    grid=(NBLK,),
    in_specs=[pl.BlockSpec((BSZ, S), lambda i: (i, 0)),
              pl.BlockSpec((BSZ, 1), lambda i: (i, 0))],
    out_specs=pl.BlockSpec((BSZ, WPAD), lambda i: (i, 0)),
    out_shape=jax.ShapeDtypeStruct((BP, WPAD), jnp.float32),
)


T = (NCHUNK + NW - 1) // NW
NIDX = R * S
_STREAMS = []
_off = 0
while _off < NIDX:
    _sz = min(128, NIDX - _off)
    _STREAMS.append((_off, _sz))
    _off += _sz


def _sc_body(feat_hbm, neigh_hbm, nodes_hbm, w_hbm, out_hbm,
             idx0, idx1, w0, w1, rows0, rows1, out0, out1,
             semg0, semg1, semi0, semi1, semw0, semw1, semo0, semo1):
    wid = lax.axis_index("s") * NC + lax.axis_index("c")
    idx = (idx0, idx1)
    wv_ = (w0, w1)
    rows = (rows0, rows1)
    outv = (out0, out1)
    semg = (semg0, semg1)
    semi = (semi0, semi1)
    semw = (semw0, semw1)
    semo = (semo0, semo1)

    def c_of(t):
        return jnp.minimum(wid + t * NW, NCHUNK - 1)

    def issue_idx(t, p):
        c = c_of(t)
        pltpu.async_copy(
            neigh_hbm.at[pl.ds(c * NIDX, NIDX)],
            idx[p].at[pl.ds(0, NIDX)], semi[p])
        pltpu.async_copy(
            nodes_hbm.at[pl.ds(c * R, R)],
            idx[p].at[pl.ds(NIDX, R)], semi[p])

    def wait_idx(p):
        pltpu.make_async_copy(
            neigh_hbm.at[pl.ds(0, NIDX + R)], idx[p], semi[p]).wait()

    def issue_w(t, p):
        pltpu.async_copy(w_hbm.at[pl.ds(c_of(t) * R, R)], wv_[p], semw[p])

    def wait_w(p):
        pltpu.make_async_copy(w_hbm.at[pl.ds(0, R)], wv_[p], semw[p]).wait()

    def issue_gather(p):
        for s, n in _STREAMS:
            pltpu.async_copy(feat_hbm.at[idx[p].at[pl.ds(s, n)]],
                             rows[p].at[pl.ds(s, n)], semg[p])
        pltpu.async_copy(feat_hbm.at[idx[p].at[pl.ds(NIDX, R)]],
                         rows[p].at[pl.ds(NIDX, R)], semg[p])

    def wait_gather(p):
        pltpu.make_async_copy(
            feat_hbm.at[pl.ds(0, NIDX + R)], rows[p], semg[p]).wait()

    def issue_out(t, p):
        pltpu.async_copy(outv[p], out_hbm.at[pl.ds(c_of(t) * R, R)], semo[p])

    def wait_out(p):
        pltpu.make_async_copy(
            outv[p], out_hbm.at[pl.ds(0, R)], semo[p]).wait()

    def compute(p):
        rows_v = rows[p]
        w_v = wv_[p]
        out_v = outv[p]
        sidx = jnp.full((L,), S - L, jnp.int32)
        for r in range(R):
            wlo = w_v[r, pl.ds(0, L)]
            whi = w_v[r, pl.ds(L, L)]
            def j_body(j, accs, wlo=wlo, whi=whi, r=r):
                jm = jnp.full((L,), j & (L - 1), jnp.int32)
                wvec = jnp.where(j < L,
                                 wlo.at[jm].get(mode="promise_in_bounds"),
                                 whi.at[jm].get(mode="promise_in_bounds"))
                row = r * S + j
                return tuple(
                    accs[k] + wvec * rows_v[row, pl.ds(k * L, L)]
                    for k in range(D // L))
            accs = lax.fori_loop(
                0, S, j_body,
                tuple(jnp.zeros((L,), jnp.float32) for _ in range(D // L)))
            wself = whi.at[sidx].get(mode="promise_in_bounds")
            for k in range(D // L):
                out_v[r, pl.ds(k * L, L)] = (
                    accs[k] + wself * rows_v[NIDX + r, pl.ds(k * L, L)])

    def step(t, p, first, second):
        q = 1 - p
        wait_gather(p)
        wait_idx(q)
        issue_gather(q)
        issue_idx(t + 2, p)
        if not (first or second):
            wait_out(p)
        wait_w(p)
        compute(p)
        issue_out(t, p)
        issue_w(t + 2, p)

    issue_idx(0, 0)
    issue_idx(1, 1)
    issue_w(0, 0)
    issue_w(1, 1)
    wait_idx(0)
    issue_gather(0)

    step(0, 0, True, False)
    step(1, 1, False, True)

    def pair_body(u, carry):
        t = 2 + 2 * u
        step(t, 0, False, False)
        step(t + 1, 1, False, False)
        return carry

    lax.fori_loop(0, (T - 2) // 2, pair_body, 0)
    if T % 2 == 1:
        step(T - 1, 0, False, False)

    pl_ = (T - 1) % 2
    ql_ = 1 - pl_
    wait_gather(ql_)
    wait_idx(pl_)
    wait_w(pl_)
    wait_out(pl_)
    wait_out(ql_)


@functools.cache
def _sc_aggregate():
    return functools.partial(
        pl.kernel,
        out_type=jax.ShapeDtypeStruct((B, D), jnp.float32),
        mesh=plsc.VectorSubcoreMesh(
            core_axis_name="c", subcore_axis_name="s",
            num_cores=NC, num_subcores=NS),
        scratch_types=(
            [pltpu.VMEM((NIDX + R,), jnp.int32)] * 2
            + [pltpu.VMEM((R, WPAD), jnp.float32)] * 2
            + [pltpu.VMEM((NIDX + R, D), jnp.float32)] * 2
            + [pltpu.VMEM((R, D), jnp.float32)] * 2
            + [pltpu.SemaphoreType.DMA] * 8
        ),
    )(_sc_body)


def kernel(features, nodes, neigh_idx):
    nodes = nodes.astype(jnp.int32)
    neigh_idx = neigh_idx.astype(jnp.int32)
    w = _tc_weights(neigh_idx, nodes[:, None])
    return _sc_aggregate()(
        features, neigh_idx.reshape(B * S), nodes, w)


# speedup vs baseline: 3.5531x; 1.0559x over previous
import functools

import jax
import jax.numpy as jnp
from jax import lax
from jax.experimental import pallas as pl
from jax.experimental.pallas import tpu as pltpu
from jax.experimental.pallas import tpu_sc as plsc

B = 20000
S = 25
S1 = S + 1
D = 128
WPAD = 32
L = 16

NC = 2
NS = 16
NW = NC * NS

R = 16
NCHUNK = B // R

BSZ = 512
NBLK = (B + BSZ - 1) // BSZ
BP = NBLK * BSZ


def _weights_body(neigh_ref, nodes_ref, w_ref):
    nT = jnp.swapaxes(neigh_ref[...], 0, 1)
    self_row = nodes_ref[...][None, :]
    a = jnp.concatenate([nT, self_row], axis=0)
    eq = a[None, :, :] == a[:, None, :]
    jj = lax.broadcasted_iota(jnp.int32, (S1, S1, 1), 0)
    kk = lax.broadcasted_iota(jnp.int32, (S1, S1, 1), 1)
    dup = jnp.any(eq & (kk < jj), axis=1)
    valid = (~dup).astype(jnp.float32)
    num = jnp.sum(valid, axis=0, keepdims=True)
    w = valid / num
    wp = jnp.concatenate(
        [w, jnp.zeros((WPAD - S1, BSZ), jnp.float32)], axis=0)
    w_ref[...] = jnp.swapaxes(wp, 0, 1)


_tc_weights = pl.pallas_call(
    _weights_body,
    grid=(NBLK,),
    in_specs=[pl.BlockSpec((BSZ, S), lambda i: (i, 0)),
              pl.BlockSpec((BSZ,), lambda i: (i,))],
    out_specs=pl.BlockSpec((BSZ, WPAD), lambda i: (i, 0)),
    out_shape=jax.ShapeDtypeStruct((BP, WPAD), jnp.float32),
)


T = (NCHUNK + NW - 1) // NW
NIDX = R * S
_STREAMS = []
_off = 0
while _off < NIDX:
    _sz = min(128, NIDX - _off)
    _STREAMS.append((_off, _sz))
    _off += _sz


def _sc_body(feat_hbm, neigh_hbm, nodes_hbm, w_hbm, out_hbm,
             idx0, idx1, w0, w1, rows0, rows1, out0, out1,
             semg0, semg1, semi0, semi1, semw0, semw1, semo0, semo1):
    wid = lax.axis_index("s") * NC + lax.axis_index("c")
    idx = (idx0, idx1)
    wv_ = (w0, w1)
    rows = (rows0, rows1)
    outv = (out0, out1)
    semg = (semg0, semg1)
    semi = (semi0, semi1)
    semw = (semw0, semw1)
    semo = (semo0, semo1)

    def c_of(t):
        return jnp.minimum(wid + t * NW, NCHUNK - 1)

    def issue_idx(t, p):
        c = c_of(t)
        pltpu.async_copy(
            neigh_hbm.at[pl.ds(c * NIDX, NIDX)],
            idx[p].at[pl.ds(0, NIDX)], semi[p])
        pltpu.async_copy(
            nodes_hbm.at[pl.ds(c * R, R)],
            idx[p].at[pl.ds(NIDX, R)], semi[p])

    def wait_idx(p):
        pltpu.make_async_copy(
            neigh_hbm.at[pl.ds(0, NIDX + R)], idx[p], semi[p]).wait()

    def issue_w(t, p):
        pltpu.async_copy(w_hbm.at[pl.ds(c_of(t) * R, R)], wv_[p], semw[p])

    def wait_w(p):
        pltpu.make_async_copy(w_hbm.at[pl.ds(0, R)], wv_[p], semw[p]).wait()

    def issue_gather(p):
        for s, n in _STREAMS:
            pltpu.async_copy(feat_hbm.at[idx[p].at[pl.ds(s, n)]],
                             rows[p].at[pl.ds(s, n)], semg[p])
        pltpu.async_copy(feat_hbm.at[idx[p].at[pl.ds(NIDX, R)]],
                         rows[p].at[pl.ds(NIDX, R)], semg[p])

    def wait_gather(p):
        pltpu.make_async_copy(
            feat_hbm.at[pl.ds(0, NIDX + R)], rows[p], semg[p]).wait()

    def issue_out(t, p):
        pltpu.async_copy(outv[p], out_hbm.at[pl.ds(c_of(t) * R, R)], semo[p])

    def wait_out(p):
        pltpu.make_async_copy(
            outv[p], out_hbm.at[pl.ds(0, R)], semo[p]).wait()

    def compute(p):
        rows_v = rows[p]
        w_v = wv_[p]
        out_v = outv[p]
        sidx = jnp.full((L,), S - L, jnp.int32)
        for r in range(R):
            wlo = w_v[r, pl.ds(0, L)]
            whi = w_v[r, pl.ds(L, L)]
            def j_body(j, accs, wlo=wlo, whi=whi, r=r):
                jm = jnp.full((L,), j & (L - 1), jnp.int32)
                wvec = jnp.where(j < L,
                                 wlo.at[jm].get(mode="promise_in_bounds"),
                                 whi.at[jm].get(mode="promise_in_bounds"))
                row = r * S + j
                return tuple(
                    accs[k] + wvec * rows_v[row, pl.ds(k * L, L)]
                    for k in range(D // L))
            accs = lax.fori_loop(
                0, S, j_body,
                tuple(jnp.zeros((L,), jnp.float32) for _ in range(D // L)))
            wself = whi.at[sidx].get(mode="promise_in_bounds")
            for k in range(D // L):
                out_v[r, pl.ds(k * L, L)] = (
                    accs[k] + wself * rows_v[NIDX + r, pl.ds(k * L, L)])

    def step(t, p, first, second):
        q = 1 - p
        wait_gather(p)
        wait_idx(q)
        issue_gather(q)
        issue_idx(t + 2, p)
        if not (first or second):
            wait_out(p)
        wait_w(p)
        compute(p)
        issue_out(t, p)
        issue_w(t + 2, p)

    issue_idx(0, 0)
    issue_idx(1, 1)
    issue_w(0, 0)
    issue_w(1, 1)
    wait_idx(0)
    issue_gather(0)

    step(0, 0, True, False)
    step(1, 1, False, True)

    def pair_body(u, carry):
        t = 2 + 2 * u
        step(t, 0, False, False)
        step(t + 1, 1, False, False)
        return carry

    lax.fori_loop(0, (T - 2) // 2, pair_body, 0)
    if T % 2 == 1:
        step(T - 1, 0, False, False)

    pl_ = (T - 1) % 2
    ql_ = 1 - pl_
    wait_gather(ql_)
    wait_idx(pl_)
    wait_w(pl_)
    wait_out(pl_)
    wait_out(ql_)


@functools.cache
def _sc_aggregate():
    return functools.partial(
        pl.kernel,
        out_type=jax.ShapeDtypeStruct((B, D), jnp.float32),
        mesh=plsc.VectorSubcoreMesh(
            core_axis_name="c", subcore_axis_name="s",
            num_cores=NC, num_subcores=NS),
        scratch_types=(
            [pltpu.VMEM((NIDX + R,), jnp.int32)] * 2
            + [pltpu.VMEM((R, WPAD), jnp.float32)] * 2
            + [pltpu.VMEM((NIDX + R, D), jnp.float32)] * 2
            + [pltpu.VMEM((R, D), jnp.float32)] * 2
            + [pltpu.SemaphoreType.DMA] * 8
        ),
    )(_sc_body)


def kernel(features, nodes, neigh_idx):
    nodes = nodes.astype(jnp.int32)
    neigh_idx = neigh_idx.astype(jnp.int32)
    w = _tc_weights(neigh_idx, nodes)
    return _sc_aggregate()(
        features, neigh_idx.reshape(B * S), nodes, w)
